# Initial kernel scaffold; baseline (speedup 1.0000x reference)
import jax
import jax.numpy as jnp
from jax.experimental import pallas as pl


def kernel(x, edge_index, edge_weight, pool_weight, init_W, w_ih, w_hh, b_ih, b_hh, lin_w, lin_b):
    raise NotImplementedError("write your pallas kernel here")


# speedup vs baseline: 10.5578x; 10.5578x over previous
import functools

import jax
import jax.numpy as jnp
from jax import lax
from jax.experimental import pallas as pl
from jax.experimental.pallas import tpu as pltpu
from jax.experimental.pallas import tpu_sc as plsc

N = 10000
E = 320000
D = 128

NC = 2
NS = 16
NW = NC * NS
EPW = E // NW
CHUNK = 80
NCHUNK = EPW // CHUNK

RB = 1000
NG = N // RB

_mesh = plsc.VectorSubcoreMesh(core_axis_name="c", subcore_axis_name="s")


def _score_body(x_ref, pn_ref, o_ref):
    o_ref[...] = jnp.tanh(jnp.sum(x_ref[...] * pn_ref[...], axis=1))


def _gru_body(xt_ref, h_ref, wih_ref, whh_ref, bih_ref, bhh_ref, w_ref):
    nt = (((1,), (1,)), ((), ()))
    gi = lax.dot_general(xt_ref[...], wih_ref[...], nt,
                         preferred_element_type=jnp.float32) + bih_ref[...]
    gh = lax.dot_general(h_ref[...], whh_ref[...], nt,
                         preferred_element_type=jnp.float32) + bhh_ref[...]
    r = jax.nn.sigmoid(gi[:, 0:D] + gh[:, 0:D])
    z = jax.nn.sigmoid(gi[:, D:2 * D] + gh[:, D:2 * D])
    n = jnp.tanh(gi[:, 2 * D:] + r * gh[:, 2 * D:])
    w_ref[...] = (1.0 - z) * n + z * h_ref[...]


def _y_body(x_ref, w_ref, degtab_ref, y_ref, dinv_ref):
    deg = degtab_ref[0, :, 0:1] + degtab_ref[1, :, 0:1] + 1.0
    dinv = lax.rsqrt(deg)
    dinv_ref[...] = dinv
    xw = jnp.dot(x_ref[...], w_ref[...], preferred_element_type=jnp.float32)
    y_ref[...] = xw * dinv


def _final_body(p_ref, y_ref, dinv_ref, lw_ref, lb_ref, o_ref):
    dinv = dinv_ref[...]
    agg = (p_ref[0] + p_ref[1] + y_ref[...]) * dinv
    h = jnp.maximum(agg, 0.0)
    o_ref[...] = jnp.sum(h * lw_ref[...], axis=1, keepdims=True) + lb_ref[0, 0]


RSTAGE = 200


@functools.partial(
    pl.kernel,
    out_type=jax.ShapeDtypeStruct((NC, N, D), jnp.float32),
    mesh=_mesh,
    scratch_types=[
        pltpu.VMEM_SHARED((N, D), jnp.float32),
        pltpu.VMEM((CHUNK,), jnp.int32),
        pltpu.VMEM((CHUNK,), jnp.float32),
        pltpu.VMEM((CHUNK, D), jnp.float32),
        pltpu.VMEM((RSTAGE, D), jnp.float32),
    ],
)
def _deg_kernel(dst_hbm, ew_hbm, out_hbm, acc_sh, dstv, ewv, rowbuf, stage):
    cid = lax.axis_index("c")
    sid = lax.axis_index("s")
    wid = sid * NC + cid

    zero = jnp.zeros((16,), jnp.float32)

    @pl.loop(0, CHUNK)
    def _(i):
        for j in range(D // 16):
            rowbuf[i, pl.ds(j * 16, 16)] = zero

    @pl.when(sid < NG)
    def _zero():
        @pl.loop(0, RSTAGE)
        def _(i):
            for j in range(D // 16):
                stage[i, pl.ds(j * 16, 16)] = zero

        @pl.loop(0, RB // RSTAGE)
        def _(k):
            pltpu.sync_copy(stage,
                            acc_sh.at[pl.ds(sid * RB + k * RSTAGE, RSTAGE)])

    plsc.subcore_barrier()
    base = wid * EPW

    @pl.loop(0, NCHUNK)
    def _(c):
        off = base + c * CHUNK
        pltpu.sync_copy(dst_hbm.at[pl.ds(off, CHUNK)], dstv)
        pltpu.sync_copy(ew_hbm.at[pl.ds(off, CHUNK)], ewv)

        @pl.loop(0, CHUNK // 16)
        def _(g):
            ew16 = ewv[pl.ds(g * 16, 16)]
            for l in range(16):
                rowbuf[g * 16 + l, pl.ds(0, 16)] = jnp.full((16,), ew16[l],
                                                            jnp.float32)

        pltpu.sync_copy(rowbuf, acc_sh.at[dstv], add=True)

    plsc.subcore_barrier()

    @pl.when(sid < NG)
    def _writeback():
        @pl.loop(0, RB // RSTAGE)
        def _(k):
            r0 = sid * RB + k * RSTAGE
            pltpu.sync_copy(acc_sh.at[pl.ds(r0, RSTAGE)], stage)
            pltpu.sync_copy(stage, out_hbm.at[cid, pl.ds(r0, RSTAGE)])


@functools.partial(
    pl.kernel,
    out_type=jax.ShapeDtypeStruct((NC, N, D), jnp.float32),
    mesh=_mesh,
    scratch_types=[
        pltpu.VMEM_SHARED((N, D), jnp.float32),
        pltpu.VMEM((CHUNK,), jnp.int32),
        pltpu.VMEM((CHUNK,), jnp.int32),
        pltpu.VMEM((CHUNK,), jnp.float32),
        pltpu.VMEM((CHUNK, D), jnp.float32),
        pltpu.VMEM((RSTAGE, D), jnp.float32),
        pltpu.SemaphoreType.DMA,
    ],
)
def _row_kernel(src_hbm, dst_hbm, ew_hbm, y_hbm, out_hbm,
                acc_sh, srcv, dstv, ewv, rows, stage, sem):
    cid = lax.axis_index("c")
    sid = lax.axis_index("s")
    wid = sid * NC + cid

    zero = jnp.zeros((16,), jnp.float32)

    @pl.when(sid < NG)
    def _zero():
        @pl.loop(0, RSTAGE)
        def _(i):
            for j in range(D // 16):
                stage[i, pl.ds(j * 16, 16)] = zero

        @pl.loop(0, RB // RSTAGE)
        def _(k):
            pltpu.sync_copy(stage,
                            acc_sh.at[pl.ds(sid * RB + k * RSTAGE, RSTAGE)])

    plsc.subcore_barrier()
    base = wid * EPW

    @pl.loop(0, NCHUNK)
    def _(c):
        off = base + c * CHUNK
        pltpu.sync_copy(src_hbm.at[pl.ds(off, CHUNK)], srcv)
        pltpu.sync_copy(dst_hbm.at[pl.ds(off, CHUNK)], dstv)
        pltpu.sync_copy(ew_hbm.at[pl.ds(off, CHUNK)], ewv)
        pltpu.async_copy(y_hbm.at[srcv], rows, sem).wait()

        @pl.loop(0, CHUNK // 16)
        def _(g):
            ew16 = ewv[pl.ds(g * 16, 16)]
            for l in range(16):
                s = ew16[l]
                i = g * 16 + l
                for j in range(D // 16):
                    rows[i, pl.ds(j * 16, 16)] = rows[i, pl.ds(j * 16, 16)] * s

        pltpu.sync_copy(rows, acc_sh.at[dstv], add=True)

    plsc.subcore_barrier()

    @pl.when(sid < NG)
    def _writeback():
        @pl.loop(0, RB // RSTAGE)
        def _(k):
            r0 = sid * RB + k * RSTAGE
            pltpu.sync_copy(acc_sh.at[pl.ds(r0, RSTAGE)], stage)
            pltpu.sync_copy(stage, out_hbm.at[cid, pl.ds(r0, RSTAGE)])


def kernel(x, edge_index, edge_weight, pool_weight, init_W,
           w_ih, w_hh, b_ih, b_hh, lin_w, lin_b):
    pn = (pool_weight / jnp.linalg.norm(pool_weight)).reshape(1, D)

    score = pl.pallas_call(
        _score_body,
        out_shape=jax.ShapeDtypeStruct((N,), jnp.float32),
    )(x, pn)

    topv, perm = lax.top_k(score, D)
    X_t = x[perm] * topv[:, None]

    W = pl.pallas_call(
        _gru_body,
        out_shape=jax.ShapeDtypeStruct((D, D), jnp.float32),
    )(X_t, init_W, w_ih, w_hh, b_ih.reshape(1, 3 * D), b_hh.reshape(1, 3 * D))

    src = edge_index[0]
    dst = edge_index[1]

    degtab = _deg_kernel(dst, edge_weight)

    y, dinv = pl.pallas_call(
        _y_body,
        grid=(NG,),
        in_specs=[
            pl.BlockSpec((RB, D), lambda i: (i, 0)),
            pl.BlockSpec((D, D), lambda i: (0, 0)),
            pl.BlockSpec((NC, RB, D), lambda i: (0, i, 0)),
        ],
        out_specs=[
            pl.BlockSpec((RB, D), lambda i: (i, 0)),
            pl.BlockSpec((RB, 1), lambda i: (i, 0)),
        ],
        out_shape=[
            jax.ShapeDtypeStruct((N, D), jnp.float32),
            jax.ShapeDtypeStruct((N, 1), jnp.float32),
        ],
    )(x, W, degtab)

    P = _row_kernel(src, dst, edge_weight, y)

    out = pl.pallas_call(
        _final_body,
        grid=(NG,),
        in_specs=[
            pl.BlockSpec((NC, RB, D), lambda i: (0, i, 0)),
            pl.BlockSpec((RB, D), lambda i: (i, 0)),
            pl.BlockSpec((RB, 1), lambda i: (i, 0)),
            pl.BlockSpec((1, D), lambda i: (0, 0)),
            pl.BlockSpec((1, 1), lambda i: (0, 0)),
        ],
        out_specs=pl.BlockSpec((RB, 1), lambda i: (i, 0)),
        out_shape=jax.ShapeDtypeStruct((N, 1), jnp.float32),
    )(P, y, dinv, lin_w.reshape(1, D), lin_b.reshape(1, 1))

    return out


# speedup vs baseline: 12.0383x; 1.1402x over previous
import functools

import jax
import jax.numpy as jnp
from jax import lax
from jax.experimental import pallas as pl
from jax.experimental.pallas import tpu as pltpu
from jax.experimental.pallas import tpu_sc as plsc

N = 10000
E = 320000
D = 128

NC = 2
NS = 16
NW = NC * NS
CHUNK = 128
NCHUNK = 80
EPW = NCHUNK * CHUNK
E2 = NW * EPW

RB = 1000
NG = N // RB
RSTAGE = 200

_mesh = plsc.VectorSubcoreMesh(core_axis_name="c", subcore_axis_name="s")


def _score_body(x_ref, pn_ref, o_ref):
    o_ref[...] = jnp.tanh(jnp.sum(x_ref[...] * pn_ref[...], axis=1))


def _gru_body(xt_ref, h_ref, wih_ref, whh_ref, bih_ref, bhh_ref, w_ref):
    nt = (((1,), (1,)), ((), ()))
    gi = lax.dot_general(xt_ref[...], wih_ref[...], nt,
                         preferred_element_type=jnp.float32) + bih_ref[...]
    gh = lax.dot_general(h_ref[...], whh_ref[...], nt,
                         preferred_element_type=jnp.float32) + bhh_ref[...]
    r = jax.nn.sigmoid(gi[:, 0:D] + gh[:, 0:D])
    z = jax.nn.sigmoid(gi[:, D:2 * D] + gh[:, D:2 * D])
    n = jnp.tanh(gi[:, 2 * D:] + r * gh[:, 2 * D:])
    w_ref[...] = (1.0 - z) * n + z * h_ref[...]


def _y_body(x_ref, w_ref, degtab_ref, y_ref, dinv_ref):
    deg = degtab_ref[0, :, 0:1] + degtab_ref[1, :, 0:1] + 1.0
    dinv = lax.rsqrt(deg)
    dinv_ref[...] = dinv
    xw = jnp.dot(x_ref[...], w_ref[...], preferred_element_type=jnp.float32)
    y_ref[...] = xw * dinv


def _final_body(p_ref, y_ref, dinv_ref, lw_ref, lb_ref, o_ref):
    dinv = dinv_ref[...]
    agg = (p_ref[0] + p_ref[1] + y_ref[...]) * dinv
    h = jnp.maximum(agg, 0.0)
    o_ref[...] = jnp.sum(h * lw_ref[...], axis=1, keepdims=True) + lb_ref[0, 0]


HALF = NCHUNK // 2


@functools.partial(
    pl.kernel,
    out_type=jax.ShapeDtypeStruct((NC, N, D), jnp.float32),
    mesh=_mesh,
    scratch_types=[
        pltpu.VMEM_SHARED((N, D), jnp.float32),
        pltpu.VMEM((HALF, CHUNK), jnp.int32),
        pltpu.VMEM((HALF, CHUNK), jnp.float32),
        pltpu.VMEM((CHUNK, D), jnp.float32),
        pltpu.VMEM((CHUNK, D), jnp.float32),
        pltpu.SemaphoreType.DMA,
        pltpu.SemaphoreType.DMA,
    ],
)
def _deg_kernel(dst_hbm, ew_hbm, out_hbm, acc_sh, dsts, ews,
                rb0, rb1, sm0, sm1):
    cid = lax.axis_index("c")
    sid = lax.axis_index("s")
    wid = sid * NC + cid

    rowbuf = (rb0, rb1)
    sems = (sm0, sm1)
    zero = jnp.zeros((16,), jnp.float32)

    @pl.loop(0, CHUNK)
    def _(i):
        for j in range(D // 16):
            rb0[i, pl.ds(j * 16, 16)] = zero
            rb1[i, pl.ds(j * 16, 16)] = zero

    @pl.when(sid < NG)
    def _zero():
        for r0, sz in ((0, 128), (128, 128), (256, 128), (384, 128),
                       (512, 128), (640, 128), (768, 128), (896, 104)):
            pltpu.sync_copy(rb0.at[pl.ds(0, sz)],
                            acc_sh.at[pl.ds(sid * RB + r0, sz)])

    plsc.subcore_barrier()

    def build(r, s):
        @pl.loop(0, CHUNK // 16)
        def _(g):
            ew16 = ews[r, pl.ds(g * 16, 16)]
            for l in range(16):
                rowbuf[s][g * 16 + l, pl.ds(0, 16)] = jnp.full(
                    (16,), ew16[l], jnp.float32)

    def fire(r, s):
        pltpu.async_copy(rowbuf[s], acc_sh.at[dsts.at[r]], sems[s], add=True)

    def drain(s):
        pltpu.make_async_copy(rowbuf[s], acc_sh.at[dsts.at[0]],
                              sems[s]).wait()

    for h in range(2):
        pltpu.sync_copy(dst_hbm.at[wid, pl.ds(h * HALF, HALF)], dsts)
        pltpu.sync_copy(ew_hbm.at[wid, pl.ds(h * HALF, HALF)], ews)

        build(0, 0)
        fire(0, 0)
        build(1, 1)
        fire(1, 1)

        @pl.loop(0, (HALF - 2) // 2)
        def _(t):
            for k in range(2):
                r = 2 * t + 2 + k
                drain(k)
                build(r, k)
                fire(r, k)

        drain(0)
        drain(1)

    plsc.subcore_barrier()

    @pl.when(sid < NG)
    def _writeback():
        pltpu.sync_copy(acc_sh.at[pl.ds(sid * RB, RB)],
                        out_hbm.at[cid, pl.ds(sid * RB, RB)])


@functools.partial(
    pl.kernel,
    out_type=jax.ShapeDtypeStruct((NC, N, D), jnp.float32),
    mesh=_mesh,
    scratch_types=[
        pltpu.VMEM_SHARED((N, D), jnp.float32),
        pltpu.VMEM((HALF, CHUNK), jnp.int32),
        pltpu.VMEM((HALF, CHUNK), jnp.int32),
        pltpu.VMEM((HALF, CHUNK), jnp.float32),
        pltpu.VMEM((CHUNK, D), jnp.float32),
        pltpu.VMEM((CHUNK, D), jnp.float32),
        pltpu.SemaphoreType.DMA,
        pltpu.SemaphoreType.DMA,
        pltpu.SemaphoreType.DMA,
        pltpu.SemaphoreType.DMA,
    ],
)
def _row_kernel(src_hbm, dst_hbm, ew_hbm, y_hbm, out_hbm,
                acc_sh, srcs, dsts, ews, r0_, r1_,
                sg0, sg1, ss0, ss1):
    cid = lax.axis_index("c")
    sid = lax.axis_index("s")
    wid = sid * NC + cid

    rows = (r0_, r1_)
    semg = (sg0, sg1)
    sems = (ss0, ss1)
    zero = jnp.zeros((16,), jnp.float32)

    @pl.loop(0, CHUNK)
    def _(i):
        for j in range(D // 16):
            r0_[i, pl.ds(j * 16, 16)] = zero

    @pl.when(sid < NG)
    def _zero():
        for r0, sz in ((0, 128), (128, 128), (256, 128), (384, 128),
                       (512, 128), (640, 128), (768, 128), (896, 104)):
            pltpu.sync_copy(r0_.at[pl.ds(0, sz)],
                            acc_sh.at[pl.ds(sid * RB + r0, sz)])

    plsc.subcore_barrier()

    def gather(r, s):
        pltpu.async_copy(y_hbm.at[srcs.at[r]], rows[s], semg[s])

    def wait_gather(r, s):
        pltpu.make_async_copy(y_hbm.at[srcs.at[r]], rows[s], semg[s]).wait()

    def scale(r, s):
        @pl.loop(0, CHUNK // 16)
        def _(g):
            ew16 = ews[r, pl.ds(g * 16, 16)]
            for l in range(16):
                sv = ew16[l]
                i = g * 16 + l
                for j in range(D // 16):
                    rows[s][i, pl.ds(j * 16, 16)] = (
                        rows[s][i, pl.ds(j * 16, 16)] * sv)

    def scatter(r, s):
        pltpu.async_copy(rows[s], acc_sh.at[dsts.at[r]], sems[s], add=True)

    def wait_scatter(s):
        pltpu.make_async_copy(rows[s], acc_sh.at[dsts.at[0]], sems[s]).wait()

    for h in range(2):
        pltpu.sync_copy(src_hbm.at[wid, pl.ds(h * HALF, HALF)], srcs)
        pltpu.sync_copy(dst_hbm.at[wid, pl.ds(h * HALF, HALF)], dsts)
        pltpu.sync_copy(ew_hbm.at[wid, pl.ds(h * HALF, HALF)], ews)

        gather(0, 0)
        wait_gather(0, 0)
        scale(0, 0)
        scatter(0, 0)
        gather(1, 1)

        @pl.loop(0, (HALF - 2) // 2)
        def _(t):
            for k in range(2):
                r = 2 * t + 1 + k
                s = 1 - k
                wait_gather(r, s)
                scale(r, s)
                wait_scatter(1 - s)
                scatter(r, s)
                gather(r + 1, 1 - s)

        wait_gather(HALF - 1, 1)
        scale(HALF - 1, 1)
        scatter(HALF - 1, 1)
        wait_scatter(0)
        wait_scatter(1)

    plsc.subcore_barrier()

    @pl.when(sid < NG)
    def _writeback():
        pltpu.sync_copy(acc_sh.at[pl.ds(sid * RB, RB)],
                        out_hbm.at[cid, pl.ds(sid * RB, RB)])


def kernel(x, edge_index, edge_weight, pool_weight, init_W,
           w_ih, w_hh, b_ih, b_hh, lin_w, lin_b):
    pn = (pool_weight / jnp.linalg.norm(pool_weight)).reshape(1, D)

    score = pl.pallas_call(
        _score_body,
        out_shape=jax.ShapeDtypeStruct((N,), jnp.float32),
    )(x, pn)

    topv, perm = lax.top_k(score, D)
    X_t = x[perm] * topv[:, None]

    W = pl.pallas_call(
        _gru_body,
        out_shape=jax.ShapeDtypeStruct((D, D), jnp.float32),
    )(X_t, init_W, w_ih, w_hh, b_ih.reshape(1, 3 * D), b_hh.reshape(1, 3 * D))

    padi = jnp.zeros((2, E2 - E), jnp.int32)
    padf = jnp.zeros((E2 - E,), jnp.float32)
    ei = jnp.concatenate([edge_index, padi], axis=1)
    src3 = ei[0].reshape(NW, NCHUNK, CHUNK)
    dst3 = ei[1].reshape(NW, NCHUNK, CHUNK)
    ew3 = jnp.concatenate([edge_weight, padf]).reshape(NW, NCHUNK, CHUNK)

    degtab = _deg_kernel(dst3, ew3)

    y, dinv = pl.pallas_call(
        _y_body,
        grid=(NG,),
        in_specs=[
            pl.BlockSpec((RB, D), lambda i: (i, 0)),
            pl.BlockSpec((D, D), lambda i: (0, 0)),
            pl.BlockSpec((NC, RB, D), lambda i: (0, i, 0)),
        ],
        out_specs=[
            pl.BlockSpec((RB, D), lambda i: (i, 0)),
            pl.BlockSpec((RB, 1), lambda i: (i, 0)),
        ],
        out_shape=[
            jax.ShapeDtypeStruct((N, D), jnp.float32),
            jax.ShapeDtypeStruct((N, 1), jnp.float32),
        ],
    )(x, W, degtab)

    P = _row_kernel(src3, dst3, ew3, y)

    out = pl.pallas_call(
        _final_body,
        grid=(NG,),
        in_specs=[
            pl.BlockSpec((NC, RB, D), lambda i: (0, i, 0)),
            pl.BlockSpec((RB, D), lambda i: (i, 0)),
            pl.BlockSpec((RB, 1), lambda i: (i, 0)),
            pl.BlockSpec((1, D), lambda i: (0, 0)),
            pl.BlockSpec((1, 1), lambda i: (0, 0)),
        ],
        out_specs=pl.BlockSpec((RB, 1), lambda i: (i, 0)),
        out_shape=jax.ShapeDtypeStruct((N, 1), jnp.float32),
    )(P, y, dinv, lin_w.reshape(1, D), lin_b.reshape(1, 1))

    return out
